# baseline (speedup 1.0000x reference)
import functools

import jax
import jax.numpy as jnp
import numpy as np
from jax import lax
from jax.experimental import pallas as pl
from jax.experimental.pallas import tpu as pltpu
from jax.experimental.pallas import tpu_sc as plsc

NUM_BINS = 15
_STEP = float(np.float32(1.0) / np.float32(NUM_BINS))

N_SC = 450560
SC_WORKERS = 32
SC_CHUNK = 160


def _bucket(conf):
    d = jnp.float32(_STEP)
    b0 = (conf * 15.0).astype(jnp.int32)
    up = jnp.where(b0 >= NUM_BINS - 1, 1.0, (b0.astype(jnp.float32) + 1.0) * d)
    b1 = b0 + (conf >= up).astype(jnp.int32)
    b2 = b1 - (conf <= b1.astype(jnp.float32) * d).astype(jnp.int32)
    b2f = b2.astype(jnp.float32)
    e_lo = b2f * d
    e_hi = jnp.where(b2 >= NUM_BINS - 1, 1.0, (b2f + 1.0) * d)
    valid = (b2 >= 0) & (b2 < NUM_BINS) & (e_lo < conf) & (conf < e_hi)
    return jnp.where(valid, b2, NUM_BINS)


def _tc_kernel(n_total, pro_ref, lab_ref, pk_ref, csum_ref):
    step = pl.program_id(0)

    @pl.when(step == 0)
    def _init():
        pk_ref[...] = jnp.zeros_like(pk_ref)
        csum_ref[...] = jnp.zeros_like(csum_ref)

    p = pro_ref[...]
    bn, c = p.shape
    g = bn // c
    p3 = p.reshape(g, c, c)
    t = jnp.transpose(p3, (0, 2, 1))
    conf = jnp.max(t, axis=1)
    cls = lax.broadcasted_iota(jnp.int32, (g, c, c), 1)
    pred = jnp.min(jnp.where(t == conf[:, None, :], cls, c), axis=1)
    lab = lab_ref[0]
    packed = jnp.where(pred == lab, jnp.int32(65537), jnp.int32(1))

    binid = _bucket(conf)
    sid = (
        step * bn
        + lax.broadcasted_iota(jnp.int32, (g, c), 0) * c
        + lax.broadcasted_iota(jnp.int32, (g, c), 1)
    )
    binid = jnp.where(sid < n_total, binid, NUM_BINS)
    zero_i = jnp.zeros_like(packed)
    zero_f = jnp.zeros_like(conf)

    for b in range(NUM_BINS):
        m = binid == b
        pk_ref[b, :] += jnp.sum(jnp.where(m, packed, zero_i), axis=0)
        csum_ref[b, :] += jnp.sum(jnp.where(m, conf, zero_f), axis=0)


def _tc_call(probas, labels, n_tc):
    n, c = probas.shape
    bn = 128 * c
    nb = (n_tc + bn - 1) // bn
    labels3 = labels[: nb * bn].reshape(nb, bn // c, c)

    out_spec = pl.BlockSpec((NUM_BINS + 1, c), lambda i: (0, 0))
    return pl.pallas_call(
        functools.partial(_tc_kernel, n_tc),
        grid=(nb,),
        in_specs=[
            pl.BlockSpec((bn, c), lambda i: (i, 0)),
            pl.BlockSpec((1, bn // c, c), lambda i: (i, 0, 0)),
        ],
        out_specs=[out_spec, out_spec],
        out_shape=[
            jax.ShapeDtypeStruct((NUM_BINS + 1, c), jnp.int32),
            jax.ShapeDtypeStruct((NUM_BINS + 1, c), jnp.float32),
        ],
    )(probas, labels3)


def _sc_kernel(n_tc, pro_hbm, lab_hbm, pk_out, cs_out,
               buf0, buf1, lb0, lb1, accp, accc, sem0, sem1, lsem0, lsem1):
    c = 128
    wid = lax.axis_index("s") * 2 + lax.axis_index("c")
    rows_w = N_SC // SC_WORKERS
    base = n_tc + wid * rows_w
    nchunks = rows_w // SC_CHUNK
    bufs = (buf0, buf1)
    lbs = (lb0, lb1)
    sems = (sem0, sem1)
    lsems = (lsem0, lsem1)

    zi16 = jnp.zeros((16,), jnp.int32)
    zf16 = jnp.zeros((16,), jnp.float32)
    for i in range(16):
        accp[pl.ds(i * 16, 16)] = zi16
        accc[pl.ds(i * 16, 16)] = zf16

    def start(ci, b):
        off = base + ci * SC_CHUNK
        pltpu.make_async_copy(
            pro_hbm.at[pl.ds(off, SC_CHUNK)], bufs[b].at[:, 0:c],
            sems[b]).start()
        pltpu.make_async_copy(
            lab_hbm.at[pl.ds(off, SC_CHUNK)], lbs[b], lsems[b]).start()

    def wait(b):
        pltpu.make_async_copy(
            pro_hbm.at[pl.ds(0, SC_CHUNK)], bufs[b].at[:, 0:c],
            sems[b]).wait()
        pltpu.make_async_copy(
            lab_hbm.at[pl.ds(0, SC_CHUNK)], lbs[b], lsems[b]).wait()

    lane = lax.iota(jnp.int32, 16)

    iotas = [lax.iota(jnp.int32, 16) + 16 * k for k in range(8)]
    perms = [lane ^ s for s in (8, 4, 2, 1)]

    def _bfly(x, op):
        for p in perms:
            x = op(x, x.at[p].get(mode="promise_in_bounds"))
        return x

    def process_group(buf, lb, r0):
        labs = lb[pl.ds(r0, 16)]
        confv = jnp.zeros((16,), jnp.float32)
        idxv = jnp.zeros((16,), jnp.int32)
        for ri in range(16):
            row = r0 + ri
            vs = [buf[row, pl.ds(k * 16, 16)] for k in range(8)]
            t = [jnp.maximum(vs[2 * k], vs[2 * k + 1]) for k in range(4)]
            t = [jnp.maximum(t[0], t[1]), jnp.maximum(t[2], t[3])]
            mx = _bfly(jnp.maximum(t[0], t[1]), jnp.maximum)
            cs = [
                jnp.where(vs[k] == mx, iotas[k], jnp.int32(1024))
                for k in range(8)
            ]
            u = [jnp.minimum(cs[2 * k], cs[2 * k + 1]) for k in range(4)]
            u = [jnp.minimum(u[0], u[1]), jnp.minimum(u[2], u[3])]
            idx = _bfly(jnp.minimum(u[0], u[1]), jnp.minimum)
            keep = lane == ri
            confv = jnp.where(keep, mx, confv)
            idxv = jnp.where(keep, idx, idxv)
        packed = jnp.where(labs == idxv, jnp.int32(65537), jnp.int32(1))
        binid = _bucket(confv)
        slot = binid * 16 + lane
        plsc.addupdate_scatter(accp, [slot], packed)
        plsc.addupdate_scatter(accc, [slot], confv)

    start(0, 0)
    start(1, 1)

    @pl.loop(0, nchunks // 2)
    def _chunks(k):
        for b in range(2):
            ci = k * 2 + b
            wait(b)
            @pl.loop(0, SC_CHUNK // 32)
            def _groups(gi, _b=b):
                process_group(bufs[_b], lbs[_b], gi * 32)
                process_group(bufs[_b], lbs[_b], gi * 32 + 16)

            @pl.when(ci + 2 < nchunks)
            def _prefetch(_ci=ci, _b=b):
                start(_ci + 2, _b)

    pltpu.sync_copy(accp, pk_out.at[wid])
    pltpu.sync_copy(accc, cs_out.at[wid])


def _sc_call(probas, labels, n_tc):
    mesh = plsc.VectorSubcoreMesh(core_axis_name="c", subcore_axis_name="s")
    run = pl.kernel(
        functools.partial(_sc_kernel, n_tc),
        out_type=[
            jax.ShapeDtypeStruct((SC_WORKERS, 256), jnp.int32),
            jax.ShapeDtypeStruct((SC_WORKERS, 256), jnp.float32),
        ],
        mesh=mesh,
        scratch_types=[
            pltpu.VMEM((SC_CHUNK, 129), jnp.float32),
            pltpu.VMEM((SC_CHUNK, 129), jnp.float32),
            pltpu.VMEM((SC_CHUNK,), jnp.int32),
            pltpu.VMEM((SC_CHUNK,), jnp.int32),
            pltpu.VMEM((256,), jnp.int32),
            pltpu.VMEM((256,), jnp.float32),
            pltpu.SemaphoreType.DMA,
            pltpu.SemaphoreType.DMA,
            pltpu.SemaphoreType.DMA,
            pltpu.SemaphoreType.DMA,
        ],
        compiler_params=pltpu.CompilerParams(needs_layout_passes=False),
    )
    return run(probas, labels)


@jax.jit
def kernel(probas, labels):
    n, c = probas.shape
    n_tc = n - N_SC

    sc_pk, sc_cs = _sc_call(probas, labels, n_tc)
    tc_pk, tc_cs = _tc_call(probas, labels, n_tc)

    sc_cnt = (sc_pk & 0xFFFF).sum(axis=0).reshape(16, 16)[:NUM_BINS]
    sc_cor = (sc_pk >> 16).sum(axis=0).reshape(16, 16)[:NUM_BINS]
    sc_cs = sc_cs.sum(axis=0).reshape(16, 16)[:NUM_BINS]

    counts = (
        jnp.sum(tc_pk[:NUM_BINS] & 0xFFFF, axis=1) + jnp.sum(sc_cnt, axis=1)
    ).astype(jnp.float32)
    corrs = (
        jnp.sum(tc_pk[:NUM_BINS] >> 16, axis=1) + jnp.sum(sc_cor, axis=1)
    ).astype(jnp.float32)
    csum = jnp.sum(tc_cs[:NUM_BINS], axis=1) + jnp.sum(sc_cs, axis=1)

    denom = jnp.maximum(counts, 1.0)
    empty = counts == 0.0
    confs = jnp.where(empty, jnp.nan, csum / denom)
    accs = jnp.where(empty, jnp.nan, corrs / denom)
    return confs, accs, counts


# speedup vs baseline: 1.0555x; 1.0555x over previous
import functools

import jax
import jax.numpy as jnp
import numpy as np
from jax import lax
from jax.experimental import pallas as pl
from jax.experimental.pallas import tpu as pltpu
from jax.experimental.pallas import tpu_sc as plsc

NUM_BINS = 15
_STEP = float(np.float32(1.0) / np.float32(NUM_BINS))

N_SC = 419840
SC_WORKERS = 32
SC_CHUNK = 160


def _bucket(conf):
    d = jnp.float32(_STEP)
    b0 = (conf * 15.0).astype(jnp.int32)
    up = jnp.where(b0 >= NUM_BINS - 1, 1.0, (b0.astype(jnp.float32) + 1.0) * d)
    b1 = b0 + (conf >= up).astype(jnp.int32)
    b2 = b1 - (conf <= b1.astype(jnp.float32) * d).astype(jnp.int32)
    b2f = b2.astype(jnp.float32)
    e_lo = b2f * d
    e_hi = jnp.where(b2 >= NUM_BINS - 1, 1.0, (b2f + 1.0) * d)
    valid = (b2 >= 0) & (b2 < NUM_BINS) & (e_lo < conf) & (conf < e_hi)
    return jnp.where(valid, b2, NUM_BINS)


def _tc_kernel(n_total, pro_ref, lab_ref, pk_ref, csum_ref):
    step = pl.program_id(0)

    @pl.when(step == 0)
    def _init():
        pk_ref[...] = jnp.zeros_like(pk_ref)
        csum_ref[...] = jnp.zeros_like(csum_ref)

    p = pro_ref[...]
    bn, c = p.shape
    g = bn // c
    p3 = p.reshape(g, c, c)
    t = jnp.transpose(p3, (0, 2, 1))
    conf = jnp.max(t, axis=1)
    cls = lax.broadcasted_iota(jnp.int32, (g, c, c), 1)
    pred = jnp.min(jnp.where(t == conf[:, None, :], cls, c), axis=1)
    lab = lab_ref[0]
    packed = jnp.where(pred == lab, jnp.int32(65537), jnp.int32(1))

    binid = _bucket(conf)
    sid = (
        step * bn
        + lax.broadcasted_iota(jnp.int32, (g, c), 0) * c
        + lax.broadcasted_iota(jnp.int32, (g, c), 1)
    )
    binid = jnp.where(sid < n_total, binid, NUM_BINS)
    zero_i = jnp.zeros_like(packed)
    zero_f = jnp.zeros_like(conf)

    for b in range(NUM_BINS):
        m = binid == b
        pk_ref[b, :] += jnp.sum(jnp.where(m, packed, zero_i), axis=0)
        csum_ref[b, :] += jnp.sum(jnp.where(m, conf, zero_f), axis=0)


def _tc_call(probas, labels, n_tc):
    n, c = probas.shape
    bn = 128 * c
    nb = (n_tc + bn - 1) // bn
    labels3 = labels[: nb * bn].reshape(nb, bn // c, c)

    out_spec = pl.BlockSpec((NUM_BINS + 1, c), lambda i: (0, 0))
    return pl.pallas_call(
        functools.partial(_tc_kernel, n_tc),
        grid=(nb,),
        in_specs=[
            pl.BlockSpec((bn, c), lambda i: (i, 0)),
            pl.BlockSpec((1, bn // c, c), lambda i: (i, 0, 0)),
        ],
        out_specs=[out_spec, out_spec],
        out_shape=[
            jax.ShapeDtypeStruct((NUM_BINS + 1, c), jnp.int32),
            jax.ShapeDtypeStruct((NUM_BINS + 1, c), jnp.float32),
        ],
    )(probas, labels3)


def _sc_kernel(n_tc, pro_hbm, lab_hbm, pk_out, cs_out,
               buf0, buf1, lb0, lb1, accp, accc, sem0, sem1, lsem0, lsem1):
    c = 128
    wid = lax.axis_index("s") * 2 + lax.axis_index("c")
    rows_w = N_SC // SC_WORKERS
    base = n_tc + wid * rows_w
    nchunks = rows_w // SC_CHUNK
    bufs = (buf0, buf1)
    lbs = (lb0, lb1)
    sems = (sem0, sem1)
    lsems = (lsem0, lsem1)

    zi16 = jnp.zeros((16,), jnp.int32)
    zf16 = jnp.zeros((16,), jnp.float32)
    for i in range(16):
        accp[pl.ds(i * 16, 16)] = zi16
        accc[pl.ds(i * 16, 16)] = zf16

    def start(ci, b):
        off = base + ci * SC_CHUNK
        pltpu.make_async_copy(
            pro_hbm.at[pl.ds(off, SC_CHUNK)], bufs[b].at[:, 0:c],
            sems[b]).start()
        pltpu.make_async_copy(
            lab_hbm.at[pl.ds(off, SC_CHUNK)], lbs[b], lsems[b]).start()

    def wait(b):
        pltpu.make_async_copy(
            pro_hbm.at[pl.ds(0, SC_CHUNK)], bufs[b].at[:, 0:c],
            sems[b]).wait()
        pltpu.make_async_copy(
            lab_hbm.at[pl.ds(0, SC_CHUNK)], lbs[b], lsems[b]).wait()

    lane = lax.iota(jnp.int32, 16)

    iotas = [lax.iota(jnp.int32, 16) + 16 * k for k in range(8)]
    perms = [lane ^ s for s in (8, 4, 2, 1)]

    def _bfly(x, op):
        for p in perms:
            x = op(x, x.at[p].get(mode="promise_in_bounds"))
        return x

    def process_group(buf, lb, r0):
        labs = lb[pl.ds(r0, 16)]
        confv = jnp.zeros((16,), jnp.float32)
        idxv = jnp.zeros((16,), jnp.int32)
        for ri in range(16):
            row = r0 + ri
            vs = [buf[row, pl.ds(k * 16, 16)] for k in range(8)]
            t = [jnp.maximum(vs[2 * k], vs[2 * k + 1]) for k in range(4)]
            t = [jnp.maximum(t[0], t[1]), jnp.maximum(t[2], t[3])]
            mx = _bfly(jnp.maximum(t[0], t[1]), jnp.maximum)
            cs = [
                jnp.where(vs[k] == mx, iotas[k], jnp.int32(1024))
                for k in range(8)
            ]
            u = [jnp.minimum(cs[2 * k], cs[2 * k + 1]) for k in range(4)]
            u = [jnp.minimum(u[0], u[1]), jnp.minimum(u[2], u[3])]
            idx = _bfly(jnp.minimum(u[0], u[1]), jnp.minimum)
            keep = lane == ri
            confv = jnp.where(keep, mx, confv)
            idxv = jnp.where(keep, idx, idxv)
        packed = jnp.where(labs == idxv, jnp.int32(65537), jnp.int32(1))
        binid = _bucket(confv)
        slot = binid * 16 + lane
        plsc.addupdate_scatter(accp, [slot], packed)
        plsc.addupdate_scatter(accc, [slot], confv)

    start(0, 0)
    start(1, 1)

    @pl.loop(0, nchunks // 2)
    def _chunks(k):
        for b in range(2):
            ci = k * 2 + b
            wait(b)
            @pl.loop(0, SC_CHUNK // 32)
            def _groups(gi, _b=b):
                process_group(bufs[_b], lbs[_b], gi * 32)
                process_group(bufs[_b], lbs[_b], gi * 32 + 16)

            @pl.when(ci + 2 < nchunks)
            def _prefetch(_ci=ci, _b=b):
                start(_ci + 2, _b)

    pltpu.sync_copy(accp, pk_out.at[wid])
    pltpu.sync_copy(accc, cs_out.at[wid])


def _sc_call(probas, labels, n_tc):
    mesh = plsc.VectorSubcoreMesh(core_axis_name="c", subcore_axis_name="s")
    run = pl.kernel(
        functools.partial(_sc_kernel, n_tc),
        out_type=[
            jax.ShapeDtypeStruct((SC_WORKERS, 256), jnp.int32),
            jax.ShapeDtypeStruct((SC_WORKERS, 256), jnp.float32),
        ],
        mesh=mesh,
        scratch_types=[
            pltpu.VMEM((SC_CHUNK, 129), jnp.float32),
            pltpu.VMEM((SC_CHUNK, 129), jnp.float32),
            pltpu.VMEM((SC_CHUNK,), jnp.int32),
            pltpu.VMEM((SC_CHUNK,), jnp.int32),
            pltpu.VMEM((256,), jnp.int32),
            pltpu.VMEM((256,), jnp.float32),
            pltpu.SemaphoreType.DMA,
            pltpu.SemaphoreType.DMA,
            pltpu.SemaphoreType.DMA,
            pltpu.SemaphoreType.DMA,
        ],
        compiler_params=pltpu.CompilerParams(needs_layout_passes=False),
    )
    return run(probas, labels)


@jax.jit
def kernel(probas, labels):
    n, c = probas.shape
    n_tc = n - N_SC

    sc_pk, sc_cs = _sc_call(probas, labels, n_tc)
    tc_pk, tc_cs = _tc_call(probas, labels, n_tc)

    sc_cnt = (sc_pk & 0xFFFF).sum(axis=0).reshape(16, 16)[:NUM_BINS]
    sc_cor = (sc_pk >> 16).sum(axis=0).reshape(16, 16)[:NUM_BINS]
    sc_cs = sc_cs.sum(axis=0).reshape(16, 16)[:NUM_BINS]

    counts = (
        jnp.sum(tc_pk[:NUM_BINS] & 0xFFFF, axis=1) + jnp.sum(sc_cnt, axis=1)
    ).astype(jnp.float32)
    corrs = (
        jnp.sum(tc_pk[:NUM_BINS] >> 16, axis=1) + jnp.sum(sc_cor, axis=1)
    ).astype(jnp.float32)
    csum = jnp.sum(tc_cs[:NUM_BINS], axis=1) + jnp.sum(sc_cs, axis=1)

    denom = jnp.maximum(counts, 1.0)
    empty = counts == 0.0
    confs = jnp.where(empty, jnp.nan, csum / denom)
    accs = jnp.where(empty, jnp.nan, corrs / denom)
    return confs, accs, counts
